# baseline (speedup 1.0000x reference)
import functools

import jax
import jax.numpy as jnp
from jax.experimental import pallas as pl
from jax.experimental.pallas import tpu as pltpu

BLOCK_M = 2048


def _fused_kernel(x_ref, w_ref, b_ref, alpha_ref, beta_ref, o_ref, w_scr):
    alpha = alpha_ref[:]
    beta = beta_ref[:]
    w_rec = (alpha[:, None] * w_ref[:, :] + beta[:, None]).astype(jnp.bfloat16)
    del w_scr
    acc = jax.lax.dot_general(
        x_ref[:, :].astype(jnp.bfloat16), w_rec,
        dimension_numbers=(((1,), (1,)), ((), ())),
        preferred_element_type=jnp.float32,
    )
    o_ref[:, :] = jnp.maximum(acc + b_ref[:][None, :], 0.0)


@jax.jit
def kernel(x, W, b, alpha, beta):
    B, S, D = x.shape
    H = W.shape[0]
    M = B * S
    x2 = x.reshape(M, D)

    grid = (M // BLOCK_M,)
    out = pl.pallas_call(
        _fused_kernel,
        grid=grid,
        in_specs=[
            pl.BlockSpec((BLOCK_M, D), lambda i: (i, 0)),
            pl.BlockSpec((H, D), lambda i: (0, 0)),
            pl.BlockSpec((H,), lambda i: (0,)),
            pl.BlockSpec((H,), lambda i: (0,)),
            pl.BlockSpec((H,), lambda i: (0,)),
        ],
        out_specs=pl.BlockSpec((BLOCK_M, H), lambda i: (i, 0)),
        out_shape=jax.ShapeDtypeStruct((M, H), jnp.float32),
        scratch_shapes=[pltpu.VMEM((H, D), jnp.bfloat16)],
        compiler_params=pltpu.CompilerParams(
            dimension_semantics=("parallel",),
        ),
    )(x2, W, b, alpha, beta)
    return out.reshape(B, S, H)


# speedup vs baseline: 1.0258x; 1.0258x over previous
import functools

import jax
import jax.numpy as jnp
from jax.experimental import pallas as pl
from jax.experimental.pallas import tpu as pltpu

BLOCK_M = 3072


def _fused_kernel(x_ref, w_ref, b_ref, alpha_ref, beta_ref, o_ref, w_scr):
    alpha = alpha_ref[:]
    beta = beta_ref[:]
    w_rec = (alpha[:, None] * w_ref[:, :] + beta[:, None]).astype(jnp.bfloat16)
    del w_scr
    acc = jax.lax.dot_general(
        x_ref[:, :].astype(jnp.bfloat16), w_rec,
        dimension_numbers=(((1,), (1,)), ((), ())),
        preferred_element_type=jnp.float32,
    )
    o_ref[:, :] = jnp.maximum(acc + b_ref[:][None, :], 0.0)


@jax.jit
def kernel(x, W, b, alpha, beta):
    B, S, D = x.shape
    H = W.shape[0]
    M = B * S
    x2 = x.reshape(M, D)

    grid = (pl.cdiv(M, BLOCK_M),)
    out = pl.pallas_call(
        _fused_kernel,
        grid=grid,
        in_specs=[
            pl.BlockSpec((BLOCK_M, D), lambda i: (i, 0)),
            pl.BlockSpec((H, D), lambda i: (0, 0)),
            pl.BlockSpec((H,), lambda i: (0,)),
            pl.BlockSpec((H,), lambda i: (0,)),
            pl.BlockSpec((H,), lambda i: (0,)),
        ],
        out_specs=pl.BlockSpec((BLOCK_M, H), lambda i: (i, 0)),
        out_shape=jax.ShapeDtypeStruct((M, H), jnp.float32),
        scratch_shapes=[pltpu.VMEM((H, D), jnp.bfloat16)],
        compiler_params=pltpu.CompilerParams(
            dimension_semantics=("parallel",),
        ),
    )(x2, W, b, alpha, beta)
    return out.reshape(B, S, H)
